# Initial kernel scaffold; baseline (speedup 1.0000x reference)
import jax
import jax.numpy as jnp
from jax.experimental import pallas as pl


def kernel(x, cam, lbs_weights, verts_transform, posed_verts):
    raise NotImplementedError("write your pallas kernel here")


# speedup vs baseline: 49.9128x; 49.9128x over previous
import functools

import jax
import jax.numpy as jnp
from jax import lax
from jax.experimental import pallas as pl
from jax.experimental.pallas import tpu as pltpu
from jax.experimental.pallas import tpu_sc as plsc

KNN = 6
CN = 128
NVP = 10496
JP = 64
FR = 128
PAD_COORD = 1.0e18
BIG = 3.0e38

SC_CORES = 2
SC_SUBCORES = 16
NW = SC_CORES * SC_SUBCORES
GROW = 128


def _knn_body(x_ref, cam_ref, pvt_ref, posed_ref, dist_ref, idx_ref):
    scale = cam_ref[0, 0, 0]
    tx = cam_ref[0, 0, 1]
    ty = cam_ref[0, 0, 2]
    xb = x_ref[0]
    ax1 = lax.broadcasted_iota(jnp.int32, (CN, 3), 1)
    tvec = jnp.where(ax1 == 0, tx, jnp.where(ax1 == 1, ty, 0.0))
    q = xb / scale - tvec
    posed_ref[0] = q
    pvt = pvt_ref[0]
    p2 = jnp.sum(pvt * pvt, axis=0)
    qq = jnp.sum(q * q, axis=1)
    d2 = (qq[:, None] + p2[None, :]
          - 2.0 * jnp.dot(q, pvt, preferred_element_type=jnp.float32))
    d2 = jnp.maximum(d2, 0.0)
    col = lax.broadcasted_iota(jnp.int32, (CN, NVP), 1)
    for k in range(KNN):
        m = jnp.min(d2, axis=1, keepdims=True)
        im = jnp.min(jnp.where(d2 == m, col, NVP), axis=1, keepdims=True)
        dist_ref[0, :, k] = m[:, 0]
        idx_ref[0, :, k] = im[:, 0]
        d2 = jnp.where(col == im, BIG, d2)


def _gather_body(tab_hbm, idx_hbm, out_hbm, idx_v, rows_v, sem):
    wid = lax.axis_index("s") * SC_CORES + lax.axis_index("c")
    per_w = idx_v.shape[0]
    base = pl.multiple_of(wid * per_w, 8)
    pltpu.sync_copy(idx_hbm.at[pl.ds(base, per_w)], idx_v)

    def step(i, carry):
        o = pl.multiple_of(i * GROW, 8)
        cp = pltpu.async_copy(tab_hbm.at[idx_v.at[pl.ds(o, GROW)]],
                              rows_v, sem)
        cp.wait()
        dst = pl.multiple_of(base + o, 8)
        pltpu.sync_copy(rows_v, out_hbm.at[pl.ds(dst, GROW)])
        return carry

    lax.fori_loop(0, per_w // GROW, step, 0)


def _combine_body(rows_ref, dist_ref, posed_ref, out_ref):
    rows = rows_ref[...]
    dist = dist_ref[...]
    p = posed_ref[...]
    l0 = rows[:, 0:JP]
    ws = []
    for k in range(KNN):
        if k == 0:
            conf = jnp.ones((rows.shape[0],), jnp.float32)
        else:
            s = jnp.sum(jnp.abs(rows[:, k * FR:k * FR + JP] - l0), axis=1)
            conf = (jnp.exp(-s / 0.02) > 0.9).astype(jnp.float32)
        ws.append(jnp.exp(-dist[:, k]) * conf)
    wsum = ws[0] + ws[1] + ws[2] + ws[3] + ws[4] + ws[5]
    t = jnp.zeros((rows.shape[0], 16), jnp.float32)
    for k in range(KNN):
        wn = (ws[k] / wsum)[:, None]
        t = t + wn * rows[:, k * FR + JP:k * FR + JP + 16]
    outs = []
    for i in range(3):
        outs.append((t[:, 4 * i] * p[:, 0] + t[:, 4 * i + 1] * p[:, 1]
                     + t[:, 4 * i + 2] * p[:, 2] + t[:, 4 * i + 3])[:, None])
    out_ref[...] = jnp.concatenate(outs, axis=1)


def kernel(x, cam, lbs_weights, verts_transform, posed_verts):
    B, N, _ = x.shape
    Nv, J = lbs_weights.shape
    M = B * N
    M6 = M * KNN

    pv = jnp.pad(posed_verts, ((0, 0), (0, NVP - Nv), (0, 0)),
                 constant_values=PAD_COORD)
    pvt = pv.transpose(0, 2, 1)

    posed, dists, idx = pl.pallas_call(
        _knn_body,
        grid=(B, N // CN),
        in_specs=[
            pl.BlockSpec((1, CN, 3), lambda b, c: (b, c, 0)),
            pl.BlockSpec((1, 1, 3), lambda b, c: (b, 0, 0)),
            pl.BlockSpec((1, 3, NVP), lambda b, c: (b, 0, 0)),
        ],
        out_specs=[
            pl.BlockSpec((1, CN, 3), lambda b, c: (b, c, 0)),
            pl.BlockSpec((1, CN, KNN), lambda b, c: (b, c, 0)),
            pl.BlockSpec((1, CN, KNN), lambda b, c: (b, c, 0)),
        ],
        out_shape=[
            jax.ShapeDtypeStruct((B, N, 3), jnp.float32),
            jax.ShapeDtypeStruct((B, N, KNN), jnp.float32),
            jax.ShapeDtypeStruct((B, N, KNN), jnp.int32),
        ],
    )(x, cam.reshape(B, 1, 3), pvt)

    off = (jnp.arange(B, dtype=jnp.int32) * Nv)[:, None, None]
    idx_t = (idx + off).reshape(-1)
    lbs_pad = jnp.pad(lbs_weights, ((0, 0), (0, JP - J)))
    fused = jnp.concatenate(
        [jnp.broadcast_to(lbs_pad[None], (B, Nv, JP)),
         verts_transform.reshape(B, Nv, 16),
         jnp.zeros((B, Nv, FR - JP - 16), jnp.float32)],
        axis=-1).reshape(B * Nv, FR)

    per_w = M6 // NW
    mesh = plsc.VectorSubcoreMesh(core_axis_name="c", subcore_axis_name="s")
    gather = functools.partial(
        pl.kernel,
        mesh=mesh,
        out_type=jax.ShapeDtypeStruct((M6, FR), jnp.float32),
        scratch_types=[
            pltpu.VMEM((per_w,), jnp.int32),
            pltpu.VMEM((GROW, FR), jnp.float32),
            pltpu.SemaphoreType.DMA,
        ],
    )(_gather_body)
    rows = gather(fused, idx_t)

    BLK = 1024
    cano = pl.pallas_call(
        _combine_body,
        grid=(M // BLK,),
        in_specs=[
            pl.BlockSpec((BLK, KNN * FR), lambda i: (i, 0)),
            pl.BlockSpec((BLK, KNN), lambda i: (i, 0)),
            pl.BlockSpec((BLK, 3), lambda i: (i, 0)),
        ],
        out_specs=pl.BlockSpec((BLK, 3), lambda i: (i, 0)),
        out_shape=jax.ShapeDtypeStruct((M, 3), jnp.float32),
    )(rows.reshape(M, KNN * FR), dists.reshape(M, KNN), posed.reshape(M, 3))
    return cano.reshape(B, N, 3)


# speedup vs baseline: 55.3981x; 1.1099x over previous
import functools

import jax
import jax.numpy as jnp
from jax import lax
from jax.experimental import pallas as pl
from jax.experimental.pallas import tpu as pltpu
from jax.experimental.pallas import tpu_sc as plsc

KNN = 6
CN = 128
NVP = 10496
JP = 64
FR = 128
PAD_COORD = 1.0e18
BIG = 3.0e38

SC_CORES = 2
SC_SUBCORES = 16
NW = SC_CORES * SC_SUBCORES
GROW = 128


def _knn_body(x_ref, cam_ref, pvt_ref, posed_ref, dist_ref, idx_ref):
    scale = cam_ref[0, 0, 0]
    tx = cam_ref[0, 0, 1]
    ty = cam_ref[0, 0, 2]
    xb = x_ref[0]
    ax1 = lax.broadcasted_iota(jnp.int32, (CN, 3), 1)
    tvec = jnp.where(ax1 == 0, tx, jnp.where(ax1 == 1, ty, 0.0))
    q = xb / scale - tvec
    posed_ref[0] = q
    pvt = pvt_ref[0]
    p2 = jnp.sum(pvt * pvt, axis=0)
    qq = jnp.sum(q * q, axis=1)
    s = (qq[:, None] + p2[None, :]
         - 2.0 * jnp.dot(q, pvt, preferred_element_type=jnp.float32))
    s = jnp.maximum(s, 0.0)
    col = lax.broadcasted_iota(jnp.int32, (CN, NVP), 1)
    for k in range(KNN):
        m = jnp.min(s, axis=1, keepdims=True)
        eq = s == m
        im = jnp.min(jnp.where(eq, col, NVP), axis=1, keepdims=True)
        dist_ref[0, :, k] = m[:, 0]
        idx_ref[0, :, k] = im[:, 0]
        if k + 1 < KNN:
            s = jnp.where(eq, BIG, s)


def _gather_body(tab_hbm, idx_hbm, out_hbm, idx_v, rows_v, sem):
    wid = lax.axis_index("s") * SC_CORES + lax.axis_index("c")
    per_w = idx_v.shape[0]
    base = pl.multiple_of(wid * per_w, 8)
    pltpu.sync_copy(idx_hbm.at[pl.ds(base, per_w)], idx_v)

    def step(i, carry):
        o = pl.multiple_of(i * GROW, 8)
        cp = pltpu.async_copy(tab_hbm.at[idx_v.at[pl.ds(o, GROW)]],
                              rows_v, sem)
        cp.wait()
        dst = pl.multiple_of(base + o, 8)
        pltpu.sync_copy(rows_v, out_hbm.at[pl.ds(dst, GROW)])
        return carry

    lax.fori_loop(0, per_w // GROW, step, 0)


def _combine_body(rows_ref, dist_ref, posed_ref, out_ref):
    rows = rows_ref[...]
    dist = dist_ref[...]
    p = posed_ref[...]
    l0 = rows[:, 0:JP]
    ws = []
    for k in range(KNN):
        if k == 0:
            conf = jnp.ones((rows.shape[0],), jnp.float32)
        else:
            s = jnp.sum(jnp.abs(rows[:, k * FR:k * FR + JP] - l0), axis=1)
            conf = (jnp.exp(-s / 0.02) > 0.9).astype(jnp.float32)
        ws.append(jnp.exp(-dist[:, k]) * conf)
    wsum = ws[0] + ws[1] + ws[2] + ws[3] + ws[4] + ws[5]
    t = jnp.zeros((rows.shape[0], 16), jnp.float32)
    for k in range(KNN):
        wn = (ws[k] / wsum)[:, None]
        t = t + wn * rows[:, k * FR + JP:k * FR + JP + 16]
    outs = []
    for i in range(3):
        outs.append((t[:, 4 * i] * p[:, 0] + t[:, 4 * i + 1] * p[:, 1]
                     + t[:, 4 * i + 2] * p[:, 2] + t[:, 4 * i + 3])[:, None])
    out_ref[...] = jnp.concatenate(outs, axis=1)


def kernel(x, cam, lbs_weights, verts_transform, posed_verts):
    B, N, _ = x.shape
    Nv, J = lbs_weights.shape
    M = B * N
    M6 = M * KNN

    pv = jnp.pad(posed_verts, ((0, 0), (0, NVP - Nv), (0, 0)),
                 constant_values=PAD_COORD)
    pvt = pv.transpose(0, 2, 1)

    posed, dists, idx = pl.pallas_call(
        _knn_body,
        grid=(B, N // CN),
        in_specs=[
            pl.BlockSpec((1, CN, 3), lambda b, c: (b, c, 0)),
            pl.BlockSpec((1, 1, 3), lambda b, c: (b, 0, 0)),
            pl.BlockSpec((1, 3, NVP), lambda b, c: (b, 0, 0)),
        ],
        out_specs=[
            pl.BlockSpec((1, CN, 3), lambda b, c: (b, c, 0)),
            pl.BlockSpec((1, CN, KNN), lambda b, c: (b, c, 0)),
            pl.BlockSpec((1, CN, KNN), lambda b, c: (b, c, 0)),
        ],
        out_shape=[
            jax.ShapeDtypeStruct((B, N, 3), jnp.float32),
            jax.ShapeDtypeStruct((B, N, KNN), jnp.float32),
            jax.ShapeDtypeStruct((B, N, KNN), jnp.int32),
        ],
    )(x, cam.reshape(B, 1, 3), pvt)

    off = (jnp.arange(B, dtype=jnp.int32) * Nv)[:, None, None]
    idx_t = (idx + off).reshape(-1)
    lbs_pad = jnp.pad(lbs_weights, ((0, 0), (0, JP - J)))
    fused = jnp.concatenate(
        [jnp.broadcast_to(lbs_pad[None], (B, Nv, JP)),
         verts_transform.reshape(B, Nv, 16),
         jnp.zeros((B, Nv, FR - JP - 16), jnp.float32)],
        axis=-1).reshape(B * Nv, FR)

    per_w = M6 // NW
    mesh = plsc.VectorSubcoreMesh(core_axis_name="c", subcore_axis_name="s")
    gather = functools.partial(
        pl.kernel,
        mesh=mesh,
        out_type=jax.ShapeDtypeStruct((M6, FR), jnp.float32),
        scratch_types=[
            pltpu.VMEM((per_w,), jnp.int32),
            pltpu.VMEM((GROW, FR), jnp.float32),
            pltpu.SemaphoreType.DMA,
        ],
    )(_gather_body)
    rows = gather(fused, idx_t)

    BLK = 1024
    cano = pl.pallas_call(
        _combine_body,
        grid=(M // BLK,),
        in_specs=[
            pl.BlockSpec((BLK, KNN * FR), lambda i: (i, 0)),
            pl.BlockSpec((BLK, KNN), lambda i: (i, 0)),
            pl.BlockSpec((BLK, 3), lambda i: (i, 0)),
        ],
        out_specs=pl.BlockSpec((BLK, 3), lambda i: (i, 0)),
        out_shape=jax.ShapeDtypeStruct((M, 3), jnp.float32),
    )(rows.reshape(M, KNN * FR), dists.reshape(M, KNN), posed.reshape(M, 3))
    return cano.reshape(B, N, 3)
